# baseline (speedup 1.0000x reference)
import functools

import jax
import jax.numpy as jnp
from jax.experimental import pallas as pl
from jax.experimental.pallas import tpu as pltpu
from jax.experimental.pallas import tpu_sc as plsc

BATCH = 16384
NUM_CODES = 8192
CODE_DIM = 256
COMMIT_COST = 0.25

BM = 512
GATHER_W = 128


LANES = 128
NCHUNK = NUM_CODES // LANES


def _argmin_body(z_ref, cbt_ref, idx_ref, acc_ref, c2_ref):
    i = pl.program_id(0)

    @pl.when(i == 0)
    def _():
        cbt = cbt_ref[...]
        c2_ref[...] = jnp.sum(cbt * cbt, axis=0)

    z = z_ref[...]
    prod_m2 = jax.lax.dot_general(
        -2.0 * z, cbt_ref[...],
        dimension_numbers=(((1,), (0,)), ((), ())),
        preferred_element_type=jnp.float32,
        precision=jax.lax.Precision.DEFAULT,
    )
    z2 = jnp.sum(z * z, axis=1)

    run_m = jnp.full((BM, LANES), jnp.inf, jnp.float32)
    run_k = jnp.zeros((BM, LANES), jnp.int32)
    for k in range(NCHUNK):
        pk = prod_m2[:, k * LANES:(k + 1) * LANES]
        ck = c2_ref[pl.ds(k * LANES, LANES)]
        dk = (z2[:, None] + pk) + ck[None, :]
        hit = dk < run_m
        run_k = jnp.where(hit, k, run_k)
        run_m = jnp.minimum(dk, run_m)

    m = jnp.min(run_m, axis=1)
    lane = jax.lax.broadcasted_iota(jnp.int32, (BM, LANES), 1)
    cand = run_k * LANES + lane
    idx = jnp.min(jnp.where(run_m == m[:, None], cand, NUM_CODES), axis=1)
    idx_ref[0, 0, :] = idx

    part = jnp.sum(m).reshape(1, 1)

    @pl.when(i == 0)
    def _():
        acc_ref[...] = jnp.zeros((1, 1), jnp.float32)

    acc_ref[...] += part


def _tc_argmin(z_e, cbt):
    rows = z_e.shape[0]
    nb = rows // BM
    idx3, acc = pl.pallas_call(
        _argmin_body,
        grid=(nb,),
        in_specs=[
            pl.BlockSpec((BM, CODE_DIM), lambda i: (i, 0)),
            pl.BlockSpec((CODE_DIM, NUM_CODES), lambda i: (0, 0)),
        ],
        out_specs=[
            pl.BlockSpec((1, 1, BM), lambda i: (i, 0, 0)),
            pl.BlockSpec((1, 1), lambda i: (0, 0)),
        ],
        out_shape=[
            jax.ShapeDtypeStruct((nb, 1, BM), jnp.int32),
            jax.ShapeDtypeStruct((1, 1), jnp.float32),
        ],
        scratch_shapes=[pltpu.VMEM((NUM_CODES,), jnp.float32)],
    )(z_e, cbt)
    return idx3.reshape(rows), acc[0, 0]


def _sc_gather(codebook, indices):
    rows = indices.shape[0]
    mesh = plsc.VectorSubcoreMesh(core_axis_name="core",
                                  subcore_axis_name="subcore")

    @functools.partial(
        pl.kernel,
        out_type=jax.ShapeDtypeStruct((rows, CODE_DIM), codebook.dtype),
        mesh=mesh,
    )
    def gather_kernel(cb_hbm, i_hbm, o_hbm):
        def body(i_vmem, o_vmem):
            pltpu.sync_copy(cb_hbm.at[i_vmem.at[0]], o_vmem)

        pltpu.emit_pipeline(
            body,
            grid=(rows // GATHER_W,),
            in_specs=[pl.BlockSpec((1, GATHER_W), index_map=lambda i: (0, i))],
            out_specs=[pl.BlockSpec((GATHER_W, CODE_DIM),
                                    index_map=lambda i: (i, 0))],
            core_axis_name=("core", "subcore"),
            dimension_semantics=(pltpu.PARALLEL,),
        )(i_hbm, o_hbm)

    return gather_kernel(codebook, indices.reshape(1, rows))


NSPLIT = 4


def kernel(z_e, codebook):
    cbt = codebook.T
    rows = BATCH // NSPLIT
    idx_parts, zq_parts, acc_parts = [], [], []
    for s in range(NSPLIT):
        idx_s, acc_s = _tc_argmin(
            jax.lax.slice(z_e, (s * rows, 0), ((s + 1) * rows, CODE_DIM)), cbt)
        zq_parts.append(_sc_gather(codebook, idx_s))
        idx_parts.append(idx_s)
        acc_parts.append(acc_s)
    indices = jnp.concatenate(idx_parts)
    z_q = jnp.concatenate(zq_parts, axis=0)
    loss_num = sum(acc_parts)
    loss = (COMMIT_COST / (BATCH * CODE_DIM)) * loss_num
    return (z_q, indices, loss)


# speedup vs baseline: 1.0297x; 1.0297x over previous
import functools

import jax
import jax.numpy as jnp
from jax.experimental import pallas as pl
from jax.experimental.pallas import tpu as pltpu
from jax.experimental.pallas import tpu_sc as plsc

BATCH = 16384
NUM_CODES = 8192
CODE_DIM = 256
COMMIT_COST = 0.25

BM = 512
GATHER_W = 128


LANES = 128
NCHUNK = NUM_CODES // LANES


def _argmin_body(z_ref, cbt_ref, idx_ref, acc_ref, c2_ref):
    i = pl.program_id(0)

    @pl.when(i == 0)
    def _():
        cbt = cbt_ref[...]
        c2_ref[...] = jnp.sum(cbt * cbt, axis=0)

    z = z_ref[...]
    prod_m2 = jax.lax.dot_general(
        -2.0 * z, cbt_ref[...],
        dimension_numbers=(((1,), (0,)), ((), ())),
        preferred_element_type=jnp.float32,
        precision=jax.lax.Precision.DEFAULT,
    )
    z2 = jnp.sum(z * z, axis=1)

    run_m = jnp.full((BM, LANES), jnp.inf, jnp.float32)
    run_k = jnp.zeros((BM, LANES), jnp.int32)
    for k in range(NCHUNK):
        pk = prod_m2[:, k * LANES:(k + 1) * LANES]
        ck = c2_ref[pl.ds(k * LANES, LANES)]
        dk = (z2[:, None] + pk) + ck[None, :]
        hit = dk < run_m
        run_k = jnp.where(hit, k, run_k)
        run_m = jnp.minimum(dk, run_m)

    m = jnp.min(run_m, axis=1)
    lane = jax.lax.broadcasted_iota(jnp.int32, (BM, LANES), 1)
    cand = run_k * LANES + lane
    idx = jnp.min(jnp.where(run_m == m[:, None], cand, NUM_CODES), axis=1)
    idx_ref[0, 0, :] = idx

    part = jnp.sum(m).reshape(1, 1)

    @pl.when(i == 0)
    def _():
        acc_ref[...] = jnp.zeros((1, 1), jnp.float32)

    acc_ref[...] += part


def _tc_argmin(z_e, cbt):
    rows = z_e.shape[0]
    nb = rows // BM
    idx3, acc = pl.pallas_call(
        _argmin_body,
        grid=(nb,),
        in_specs=[
            pl.BlockSpec((BM, CODE_DIM), lambda i: (i, 0)),
            pl.BlockSpec((CODE_DIM, NUM_CODES), lambda i: (0, 0)),
        ],
        out_specs=[
            pl.BlockSpec((1, 1, BM), lambda i: (i, 0, 0)),
            pl.BlockSpec((1, 1), lambda i: (0, 0)),
        ],
        out_shape=[
            jax.ShapeDtypeStruct((nb, 1, BM), jnp.int32),
            jax.ShapeDtypeStruct((1, 1), jnp.float32),
        ],
        scratch_shapes=[pltpu.VMEM((NUM_CODES,), jnp.float32)],
    )(z_e, cbt)
    return idx3.reshape(rows), acc[0, 0]


def _sc_gather(codebook, indices):
    rows = indices.shape[0]
    mesh = plsc.VectorSubcoreMesh(core_axis_name="core",
                                  subcore_axis_name="subcore")

    @functools.partial(
        pl.kernel,
        out_type=jax.ShapeDtypeStruct((rows, CODE_DIM), codebook.dtype),
        mesh=mesh,
    )
    def gather_kernel(cb_hbm, i_hbm, o_hbm):
        def body(i_vmem, o_vmem):
            pltpu.sync_copy(cb_hbm.at[i_vmem.at[0]], o_vmem)

        pltpu.emit_pipeline(
            body,
            grid=(rows // GATHER_W,),
            in_specs=[pl.BlockSpec((1, GATHER_W), index_map=lambda i: (0, i))],
            out_specs=[pl.BlockSpec((GATHER_W, CODE_DIM),
                                    index_map=lambda i: (i, 0))],
            core_axis_name=("core", "subcore"),
            dimension_semantics=(pltpu.PARALLEL,),
        )(i_hbm, o_hbm)

    return gather_kernel(codebook, indices.reshape(1, rows))


NSPLIT = 2


def kernel(z_e, codebook):
    cbt = codebook.T
    rows = BATCH // NSPLIT
    idx_parts, zq_parts, acc_parts = [], [], []
    for s in range(NSPLIT):
        idx_s, acc_s = _tc_argmin(
            jax.lax.slice(z_e, (s * rows, 0), ((s + 1) * rows, CODE_DIM)), cbt)
        zq_parts.append(_sc_gather(codebook, idx_s))
        idx_parts.append(idx_s)
        acc_parts.append(acc_s)
    indices = jnp.concatenate(idx_parts)
    z_q = jnp.concatenate(zq_parts, axis=0)
    loss_num = sum(acc_parts)
    loss = (COMMIT_COST / (BATCH * CODE_DIM)) * loss_num
    return (z_q, indices, loss)


# speedup vs baseline: 1.0707x; 1.0398x over previous
import functools

import jax
import jax.numpy as jnp
from jax.experimental import pallas as pl
from jax.experimental.pallas import tpu as pltpu
from jax.experimental.pallas import tpu_sc as plsc

BATCH = 16384
NUM_CODES = 8192
CODE_DIM = 256
COMMIT_COST = 0.25

BM = 512
GATHER_W = 128


LANES = 128
NCHUNK = NUM_CODES // LANES


def _argmin_body(z_ref, cbt_ref, idx_ref, acc_ref, c2_ref):
    i = pl.program_id(0)

    @pl.when(i == 0)
    def _():
        cbt = cbt_ref[...]
        c2_ref[...] = jnp.sum(cbt * cbt, axis=0)

    z = z_ref[...]
    prod_m2 = jax.lax.dot_general(
        -2.0 * z, cbt_ref[...],
        dimension_numbers=(((1,), (0,)), ((), ())),
        preferred_element_type=jnp.float32,
        precision=jax.lax.Precision.DEFAULT,
    )
    z2 = jnp.sum(z * z, axis=1)

    def dist(k):
        pk = prod_m2[:, k * LANES:(k + 1) * LANES]
        ck = c2_ref[pl.ds(k * LANES, LANES)]
        return (z2[:, None] + pk) + ck[None, :]

    d0 = dist(0)
    d1 = dist(1)
    w01 = d1 < d0
    run_m = jnp.minimum(d0, d1)
    run_k = jnp.where(w01, 1, 0)
    for k in range(2, NCHUNK, 2):
        da = dist(k)
        db = dist(k + 1)
        w = db < da
        e = jnp.minimum(da, db)
        kl = jnp.where(w, k + 1, k)
        hit = e < run_m
        run_k = jnp.where(hit, kl, run_k)
        run_m = jnp.minimum(e, run_m)

    m = jnp.min(run_m, axis=1)
    lane = jax.lax.broadcasted_iota(jnp.int32, (BM, LANES), 1)
    cand = run_k * LANES + lane
    idx = jnp.min(jnp.where(run_m == m[:, None], cand, NUM_CODES), axis=1)
    idx_ref[0, 0, :] = idx

    part = jnp.sum(m).reshape(1, 1)

    @pl.when(i == 0)
    def _():
        acc_ref[...] = jnp.zeros((1, 1), jnp.float32)

    acc_ref[...] += part


def _tc_argmin(z_e, cbt):
    rows = z_e.shape[0]
    nb = rows // BM
    idx3, acc = pl.pallas_call(
        _argmin_body,
        grid=(nb,),
        in_specs=[
            pl.BlockSpec((BM, CODE_DIM), lambda i: (i, 0)),
            pl.BlockSpec((CODE_DIM, NUM_CODES), lambda i: (0, 0)),
        ],
        out_specs=[
            pl.BlockSpec((1, 1, BM), lambda i: (i, 0, 0)),
            pl.BlockSpec((1, 1), lambda i: (0, 0)),
        ],
        out_shape=[
            jax.ShapeDtypeStruct((nb, 1, BM), jnp.int32),
            jax.ShapeDtypeStruct((1, 1), jnp.float32),
        ],
        scratch_shapes=[pltpu.VMEM((NUM_CODES,), jnp.float32)],
    )(z_e, cbt)
    return idx3.reshape(rows), acc[0, 0]


def _sc_gather(codebook, indices):
    rows = indices.shape[0]
    mesh = plsc.VectorSubcoreMesh(core_axis_name="core",
                                  subcore_axis_name="subcore")

    @functools.partial(
        pl.kernel,
        out_type=jax.ShapeDtypeStruct((rows, CODE_DIM), codebook.dtype),
        mesh=mesh,
    )
    def gather_kernel(cb_hbm, i_hbm, o_hbm):
        def body(i_vmem, o_vmem):
            pltpu.sync_copy(cb_hbm.at[i_vmem.at[0]], o_vmem)

        pltpu.emit_pipeline(
            body,
            grid=(rows // GATHER_W,),
            in_specs=[pl.BlockSpec((1, GATHER_W), index_map=lambda i: (0, i))],
            out_specs=[pl.BlockSpec((GATHER_W, CODE_DIM),
                                    index_map=lambda i: (i, 0))],
            core_axis_name=("core", "subcore"),
            dimension_semantics=(pltpu.PARALLEL,),
        )(i_hbm, o_hbm)

    return gather_kernel(codebook, indices.reshape(1, rows))


NSPLIT = 1


def kernel(z_e, codebook):
    cbt = codebook.T
    rows = BATCH // NSPLIT
    idx_parts, zq_parts, acc_parts = [], [], []
    for s in range(NSPLIT):
        idx_s, acc_s = _tc_argmin(
            jax.lax.slice(z_e, (s * rows, 0), ((s + 1) * rows, CODE_DIM)), cbt)
        zq_parts.append(_sc_gather(codebook, idx_s))
        idx_parts.append(idx_s)
        acc_parts.append(acc_s)
    indices = jnp.concatenate(idx_parts)
    z_q = jnp.concatenate(zq_parts, axis=0)
    loss_num = sum(acc_parts)
    loss = (COMMIT_COST / (BATCH * CODE_DIM)) * loss_num
    return (z_q, indices, loss)
